# baseline (speedup 1.0000x reference)
import functools
import math

import jax
import jax.numpy as jnp
from jax import lax
from jax.experimental import pallas as pl
from jax.experimental.pallas import tpu as pltpu
from jax.experimental.pallas import tpu_sc as plsc

B = 1024
D = 10
NEG = 4
HIST = 20
TL = 15
NEWS_N = NEG + 1 + HIST + D
DIM = 128
SLOT = 1 + TL

NC, NS = 2, 16
NW = NC * NS

N_WORD = B * NEWS_N * SLOT
N_NEWS = B * NEWS_N
N_USER = B * (1 + D)

W_PER = N_WORD // NW
N_PER = N_NEWS // NW
U_PER = N_USER // NW
W_CH = 256
N_CH = 224


def _sc_gather_body(widx, nidx, uidx, wtab, ntab, utab,
                    wout, nout, uout,
                    widx_v, wbuf, nidx_v, nbuf, uidx_v, ubuf, sem):
    wid = lax.axis_index("s") * NC + lax.axis_index("c")

    wbase = wid * W_PER

    def wstep(i, carry):
        base = wbase + i * W_CH
        pltpu.sync_copy(widx.at[pl.ds(base, W_CH)], widx_v)
        pltpu.async_copy(wtab.at[widx_v], wbuf, sem).wait()
        pltpu.sync_copy(wbuf, wout.at[pl.ds(base, W_CH)])
        return carry

    lax.fori_loop(0, W_PER // W_CH, wstep, 0)

    nbase = wid * N_PER

    def nstep(i, carry):
        base = nbase + i * N_CH
        pltpu.sync_copy(nidx.at[pl.ds(base, N_CH)], nidx_v)
        pltpu.async_copy(ntab.at[nidx_v], nbuf, sem).wait()
        pltpu.sync_copy(nbuf, nout.at[pl.ds(base, N_CH)])
        return carry

    lax.fori_loop(0, N_PER // N_CH, nstep, 0)

    ubase = wid * U_PER
    pltpu.sync_copy(uidx.at[pl.ds(ubase, U_PER)], uidx_v)
    pltpu.async_copy(utab.at[uidx_v], ubuf, sem).wait()
    pltpu.sync_copy(ubuf, uout.at[pl.ds(ubase, U_PER)])


def _make_sc_gather():
    return functools.partial(
        pl.kernel,
        out_type=[
            jax.ShapeDtypeStruct((N_WORD, DIM), jnp.float32),
            jax.ShapeDtypeStruct((N_NEWS, DIM), jnp.float32),
            jax.ShapeDtypeStruct((N_USER, DIM), jnp.float32),
        ],
        mesh=plsc.VectorSubcoreMesh(
            core_axis_name="c", subcore_axis_name="s",
            num_cores=NC, num_subcores=NS),
        scratch_types=[
            pltpu.VMEM((W_CH,), jnp.int32),
            pltpu.VMEM((W_CH, DIM), jnp.float32),
            pltpu.VMEM((N_CH,), jnp.int32),
            pltpu.VMEM((N_CH, DIM), jnp.float32),
            pltpu.VMEM((U_PER,), jnp.int32),
            pltpu.VMEM((U_PER, DIM), jnp.float32),
            pltpu.SemaphoreType.DMA,
        ],
    )(_sc_gather_body)


BB = 16
IB = BB * NEWS_N
TR = IB * SLOT
NG = IB // 8
UB = BB * (1 + D)

_INV_SQRT_D = 1.0 / math.sqrt(DIM)


def _tc_body(w_ref, n_ref, u_ref, wq_ref, wk_ref, wv_ref, qp_ref, bias_ref,
             seg_ref, out_ref, q_s, k_s, v_s, p_s, s_s, info_s):
    w = w_ref[...].astype(jnp.bfloat16)
    wq = (wq_ref[...] * _INV_SQRT_D).astype(jnp.bfloat16)
    wk = wk_ref[...].astype(jnp.bfloat16)
    wv = wv_ref[...].astype(jnp.bfloat16)
    q_s[...] = jnp.dot(w, wq,
                       preferred_element_type=jnp.float32).astype(jnp.bfloat16)
    k_s[...] = jnp.dot(w, wk,
                       preferred_element_type=jnp.float32).astype(jnp.bfloat16)
    v_s[...] = jnp.dot(w, wv,
                       preferred_element_type=jnp.float32).astype(jnp.bfloat16)
    bias = bias_ref[...]
    qp = qp_ref[...]

    def smm(g, carry):
        qg = q_s[pl.ds(g * 128, 128), :]
        kg = k_s[pl.ds(g * 128, 128), :]
        s_s[pl.ds(g * 128, 128), :] = lax.dot_general(
            qg, kg, (((1,), (1,)), ((), ())),
            preferred_element_type=jnp.float32)
        return carry

    lax.fori_loop(0, NG, smm, 0, unroll=5)

    pe = jnp.exp(s_s[...].reshape(NG, 128, 128) + bias[None, :, :])
    p_s[...] = pe.astype(jnp.bfloat16).reshape(TR, DIM)

    p3 = p_s[...].reshape(NG, 128, 128)
    v4 = v_s[...].reshape(NG, 128, DIM)
    ones_b = jnp.ones((NG, 1, 128), jnp.bfloat16)
    recl = 1.0 / lax.dot_general(
        ones_b, p3, (((2,), (2,)), ((0,), (0,))),
        preferred_element_type=jnp.float32).reshape(NG, 128)
    qp_b = jnp.broadcast_to(qp.astype(jnp.bfloat16)[None], (NG, 1, DIM))
    u_b = lax.dot_general(qp_b, v4, (((2,), (2,)), ((0,), (0,))),
                          preferred_element_type=jnp.float32)
    ps = lax.dot_general(u_b.astype(jnp.bfloat16), p3,
                         (((2,), (2,)), ((0,), (0,))),
                         preferred_element_type=jnp.float32
                         ).reshape(NG, 128) * recl
    lane = lax.broadcasted_iota(jnp.int32, (NG, 128), 1)
    ae = jnp.exp(jnp.where(lane % SLOT != 0, ps, -1e30))
    den = jnp.dot(ae.astype(jnp.bfloat16), seg_ref[...],
                  preferred_element_type=jnp.float32)
    alpha = (ae / den * recl).astype(jnp.bfloat16)
    blk = lax.broadcasted_iota(jnp.int32, (8, 128), 0)
    lane8 = lax.broadcasted_iota(jnp.int32, (8, 128), 1)
    bmask = (lane8 // SLOT == blk).astype(jnp.bfloat16)
    w_alpha = alpha[:, None, :] * bmask[None]
    c_b = lax.dot_general(w_alpha, p3, (((2,), (1,)), ((0,), (0,))),
                          preferred_element_type=jnp.float32)
    info_s[...] = lax.dot_general(
        c_b.astype(jnp.bfloat16), v4, (((2,), (1,)), ((0,), (0,))),
        preferred_element_type=jnp.float32).reshape(IB, DIM)

    x = info_s[...] + n_ref[...]

    r2 = lax.broadcasted_iota(jnp.int32, (BB, IB), 0)
    c2 = lax.broadcasted_iota(jnp.int32, (BB, IB), 1)
    j = c2 - r2 * NEWS_N
    wnews = jnp.where((j >= NEG + 1) & (j < NEG + 1 + HIST), 1.0 / HIST,
                      jnp.where((j >= NEG + 1 + HIST) & (j < NEWS_N),
                                1.0 / D, 0.0))
    user_vec = jnp.dot(wnews, x, preferred_element_type=jnp.float32)

    r3 = lax.broadcasted_iota(jnp.int32, (BB, UB), 0)
    c3 = lax.broadcasted_iota(jnp.int32, (BB, UB), 1)
    ju = c3 - r3 * (1 + D)
    wuser = jnp.where(ju == 0, 1.0,
                      jnp.where((ju >= 1) & (ju < 1 + D), 1.0 / D, 0.0))
    user_vec = user_vec + jnp.dot(wuser, u_ref[...],
                                  preferred_element_type=jnp.float32)

    cand = x.reshape(BB, NEWS_N, DIM)[:, :NEG + 1, :]
    logits = jnp.sum(user_vec[:, None, :] * cand, axis=2)
    out_ref[...] = logits


def _attn_bias():
    r = jnp.arange(128)[:, None]
    c = jnp.arange(128)[None, :]
    valid = ((r // SLOT) == (c // SLOT)) & ((c % SLOT) != 0)
    return jnp.where(valid, 0.0, -1e30).astype(jnp.float32)


def _seg_mat():
    r = jnp.arange(128)[:, None]
    c = jnp.arange(128)[None, :]
    return ((r // SLOT) == (c // SLOT)).astype(jnp.bfloat16)


def _tc_forward(wrows, nrows, urows, Wq, Wk, Wv, q_pool):
    grid = (B // BB,)
    return pl.pallas_call(
        _tc_body,
        grid=grid,
        in_specs=[
            pl.BlockSpec((TR, DIM), lambda i: (i, 0)),
            pl.BlockSpec((IB, DIM), lambda i: (i, 0)),
            pl.BlockSpec((UB, DIM), lambda i: (i, 0)),
            pl.BlockSpec((DIM, DIM), lambda i: (0, 0)),
            pl.BlockSpec((DIM, DIM), lambda i: (0, 0)),
            pl.BlockSpec((DIM, DIM), lambda i: (0, 0)),
            pl.BlockSpec((1, DIM), lambda i: (0, 0)),
            pl.BlockSpec((128, 128), lambda i: (0, 0)),
            pl.BlockSpec((128, 128), lambda i: (0, 0)),
        ],
        out_specs=pl.BlockSpec((BB, NEG + 1), lambda i: (i, 0)),
        out_shape=jax.ShapeDtypeStruct((B, NEG + 1), jnp.float32),
        scratch_shapes=[
            pltpu.VMEM((TR, DIM), jnp.bfloat16),
            pltpu.VMEM((TR, DIM), jnp.bfloat16),
            pltpu.VMEM((TR, DIM), jnp.bfloat16),
            pltpu.VMEM((TR, DIM), jnp.bfloat16),
            pltpu.VMEM((TR, DIM), jnp.float32),
            pltpu.VMEM((IB, DIM), jnp.float32),
        ],
    )(wrows, nrows, urows, Wq, Wk, Wv, q_pool.reshape(1, DIM), _attn_bias(),
      _seg_mat())


def kernel(data, user_emb, news_emb, word_emb, Wq, Wk, Wv, q_pool):
    uidx = data[:, : 1 + D].reshape(-1)
    nidx = data[:, 1 + D: 1 + D + NEWS_N].reshape(-1)
    widx = data[:, 1 + D + NEWS_N:].reshape(-1)
    wrows, nrows, urows = _make_sc_gather()(widx, nidx, uidx,
                                            word_emb, news_emb, user_emb)
    return _tc_forward(wrows, nrows, urows, Wq, Wk, Wv, q_pool)


# speedup vs baseline: 1.2829x; 1.2829x over previous
import functools
import math

import jax
import jax.numpy as jnp
from jax import lax
from jax.experimental import pallas as pl
from jax.experimental.pallas import tpu as pltpu
from jax.experimental.pallas import tpu_sc as plsc

B = 1024
D = 10
NEG = 4
HIST = 20
TL = 15
NEWS_N = NEG + 1 + HIST + D
DIM = 128
SLOT = 1 + TL

NC, NS = 2, 16
NW = NC * NS

W_CH = 256
N_CH = 112


def _make_sc_gather(nb):
    n_word = nb * NEWS_N * SLOT
    n_news = nb * NEWS_N
    n_user = nb * (1 + D)
    w_per = n_word // NW
    n_per = n_news // NW
    u_per = n_user // NW

    def body(widx, nidx, uidx, wtab, ntab, utab, wout, nout, uout,
             widx_v, wbuf, nidx_v, nbuf, uidx_v, ubuf, sem):
        wid = lax.axis_index("s") * NC + lax.axis_index("c")

        wbase = wid * w_per

        def wstep(i, carry):
            base = wbase + i * W_CH
            pltpu.sync_copy(widx.at[pl.ds(base, W_CH)], widx_v)
            pltpu.async_copy(wtab.at[widx_v], wbuf, sem).wait()
            pltpu.sync_copy(wbuf, wout.at[pl.ds(base, W_CH)])
            return carry

        lax.fori_loop(0, w_per // W_CH, wstep, 0)

        nbase = wid * n_per

        def nstep(i, carry):
            base = nbase + i * N_CH
            pltpu.sync_copy(nidx.at[pl.ds(base, N_CH)], nidx_v)
            pltpu.async_copy(ntab.at[nidx_v], nbuf, sem).wait()
            pltpu.sync_copy(nbuf, nout.at[pl.ds(base, N_CH)])
            return carry

        lax.fori_loop(0, n_per // N_CH, nstep, 0)

        ubase = wid * u_per
        pltpu.sync_copy(uidx.at[pl.ds(ubase, u_per)], uidx_v)
        pltpu.async_copy(utab.at[uidx_v], ubuf, sem).wait()
        pltpu.sync_copy(ubuf, uout.at[pl.ds(ubase, u_per)])

    return functools.partial(
        pl.kernel,
        out_type=[
            jax.ShapeDtypeStruct((n_word, DIM), jnp.float32),
            jax.ShapeDtypeStruct((n_news, DIM), jnp.float32),
            jax.ShapeDtypeStruct((n_user, DIM), jnp.float32),
        ],
        mesh=plsc.VectorSubcoreMesh(
            core_axis_name="c", subcore_axis_name="s",
            num_cores=NC, num_subcores=NS),
        scratch_types=[
            pltpu.VMEM((W_CH,), jnp.int32),
            pltpu.VMEM((W_CH, DIM), jnp.float32),
            pltpu.VMEM((N_CH,), jnp.int32),
            pltpu.VMEM((N_CH, DIM), jnp.float32),
            pltpu.VMEM((u_per,), jnp.int32),
            pltpu.VMEM((u_per, DIM), jnp.float32),
            pltpu.SemaphoreType.DMA,
        ],
    )(body)


BB = 16
IB = BB * NEWS_N
TR = IB * SLOT
NG = IB // 8
UB = BB * (1 + D)

_INV_SQRT_D = 1.0 / math.sqrt(DIM)


def _tc_body(w_ref, n_ref, u_ref, wq_ref, wk_ref, wv_ref, qp_ref, bias_ref,
             seg_ref, out_ref, q_s, k_s, v_s, p_s, s_s, info_s):
    w = w_ref[...].astype(jnp.bfloat16)
    wq = (wq_ref[...] * _INV_SQRT_D).astype(jnp.bfloat16)
    wk = wk_ref[...].astype(jnp.bfloat16)
    wv = wv_ref[...].astype(jnp.bfloat16)
    q_s[...] = jnp.dot(w, wq,
                       preferred_element_type=jnp.float32).astype(jnp.bfloat16)
    k_s[...] = jnp.dot(w, wk,
                       preferred_element_type=jnp.float32).astype(jnp.bfloat16)
    v_s[...] = jnp.dot(w, wv,
                       preferred_element_type=jnp.float32).astype(jnp.bfloat16)
    bias = bias_ref[...]
    qp = qp_ref[...]

    def smm(g, carry):
        qg = q_s[pl.ds(g * 128, 128), :]
        kg = k_s[pl.ds(g * 128, 128), :]
        s_s[pl.ds(g * 128, 128), :] = lax.dot_general(
            qg, kg, (((1,), (1,)), ((), ())),
            preferred_element_type=jnp.float32)
        return carry

    lax.fori_loop(0, NG, smm, 0, unroll=5)

    pe = jnp.exp(s_s[...].reshape(NG, 128, 128) + bias[None, :, :])
    rec = 1.0 / jnp.sum(pe, axis=2, keepdims=True)
    p_s[...] = (pe * rec).astype(jnp.bfloat16).reshape(TR, DIM)

    p3 = p_s[...].reshape(NG, 128, 128)
    v4 = v_s[...].reshape(NG, 128, DIM)
    qp_b = jnp.broadcast_to(qp.astype(jnp.bfloat16)[None], (NG, 1, DIM))
    u_b = lax.dot_general(qp_b, v4, (((2,), (2,)), ((0,), (0,))),
                          preferred_element_type=jnp.float32)
    ps = lax.dot_general(u_b.astype(jnp.bfloat16), p3,
                         (((2,), (2,)), ((0,), (0,))),
                         preferred_element_type=jnp.float32
                         ).reshape(NG, 128)
    lane = lax.broadcasted_iota(jnp.int32, (NG, 128), 1)
    ae = jnp.exp(jnp.where(lane % SLOT != 0, ps, -1e30))
    den = jnp.dot(ae.astype(jnp.bfloat16), seg_ref[...],
                  preferred_element_type=jnp.float32)
    alpha = (ae / den).astype(jnp.bfloat16)
    blk = lax.broadcasted_iota(jnp.int32, (8, 128), 0)
    lane8 = lax.broadcasted_iota(jnp.int32, (8, 128), 1)
    bmask = (lane8 // SLOT == blk).astype(jnp.bfloat16)
    w_alpha = alpha[:, None, :] * bmask[None]
    c_b = lax.dot_general(w_alpha, p3, (((2,), (1,)), ((0,), (0,))),
                          preferred_element_type=jnp.float32)
    info_s[...] = lax.dot_general(
        c_b.astype(jnp.bfloat16), v4, (((2,), (1,)), ((0,), (0,))),
        preferred_element_type=jnp.float32).reshape(IB, DIM)

    x = info_s[...] + n_ref[...]

    r2 = lax.broadcasted_iota(jnp.int32, (BB, IB), 0)
    c2 = lax.broadcasted_iota(jnp.int32, (BB, IB), 1)
    j = c2 - r2 * NEWS_N
    wnews = jnp.where((j >= NEG + 1) & (j < NEG + 1 + HIST), 1.0 / HIST,
                      jnp.where((j >= NEG + 1 + HIST) & (j < NEWS_N),
                                1.0 / D, 0.0))
    user_vec = jnp.dot(wnews, x, preferred_element_type=jnp.float32)

    r3 = lax.broadcasted_iota(jnp.int32, (BB, UB), 0)
    c3 = lax.broadcasted_iota(jnp.int32, (BB, UB), 1)
    ju = c3 - r3 * (1 + D)
    wuser = jnp.where(ju == 0, 1.0,
                      jnp.where((ju >= 1) & (ju < 1 + D), 1.0 / D, 0.0))
    user_vec = user_vec + jnp.dot(wuser, u_ref[...],
                                  preferred_element_type=jnp.float32)

    cand = x.reshape(BB, NEWS_N, DIM)[:, :NEG + 1, :]
    logits = jnp.sum(user_vec[:, None, :] * cand, axis=2)
    out_ref[...] = logits


def _attn_bias():
    r = jnp.arange(128)[:, None]
    c = jnp.arange(128)[None, :]
    valid = ((r // SLOT) == (c // SLOT)) & ((c % SLOT) != 0)
    return jnp.where(valid, 0.0, -1e30).astype(jnp.float32)


def _seg_mat():
    r = jnp.arange(128)[:, None]
    c = jnp.arange(128)[None, :]
    return ((r // SLOT) == (c // SLOT)).astype(jnp.bfloat16)


def _tc_forward(wrows, nrows, urows, Wq, Wk, Wv, q_pool):
    nb = urows.shape[0] // (1 + D)
    grid = (nb // BB,)
    return pl.pallas_call(
        _tc_body,
        grid=grid,
        in_specs=[
            pl.BlockSpec((TR, DIM), lambda i: (i, 0)),
            pl.BlockSpec((IB, DIM), lambda i: (i, 0)),
            pl.BlockSpec((UB, DIM), lambda i: (i, 0)),
            pl.BlockSpec((DIM, DIM), lambda i: (0, 0)),
            pl.BlockSpec((DIM, DIM), lambda i: (0, 0)),
            pl.BlockSpec((DIM, DIM), lambda i: (0, 0)),
            pl.BlockSpec((1, DIM), lambda i: (0, 0)),
            pl.BlockSpec((128, 128), lambda i: (0, 0)),
            pl.BlockSpec((128, 128), lambda i: (0, 0)),
        ],
        out_specs=pl.BlockSpec((BB, NEG + 1), lambda i: (i, 0)),
        out_shape=jax.ShapeDtypeStruct((nb, NEG + 1), jnp.float32),
        scratch_shapes=[
            pltpu.VMEM((TR, DIM), jnp.bfloat16),
            pltpu.VMEM((TR, DIM), jnp.bfloat16),
            pltpu.VMEM((TR, DIM), jnp.bfloat16),
            pltpu.VMEM((TR, DIM), jnp.bfloat16),
            pltpu.VMEM((TR, DIM), jnp.float32),
            pltpu.VMEM((IB, DIM), jnp.float32),
        ],
    )(wrows, nrows, urows, Wq, Wk, Wv, q_pool.reshape(1, DIM), _attn_bias(),
      _seg_mat())


def kernel(data, user_emb, news_emb, word_emb, Wq, Wk, Wv, q_pool):
    nb = B // 2
    sc = _make_sc_gather(nb)
    halves = []
    for h in range(2):
        d = data[h * nb:(h + 1) * nb]
        uidx = d[:, : 1 + D].reshape(-1)
        nidx = d[:, 1 + D: 1 + D + NEWS_N].reshape(-1)
        widx = d[:, 1 + D + NEWS_N:].reshape(-1)
        halves.append(sc(widx, nidx, uidx, word_emb, news_emb, user_emb))
    outs = [_tc_forward(wr, nr, ur, Wq, Wk, Wv, q_pool)
            for wr, nr, ur in halves]
    return jnp.concatenate(outs, axis=0)


# speedup vs baseline: 1.3918x; 1.0849x over previous
import functools
import math

import jax
import jax.numpy as jnp
from jax import lax
from jax.experimental import pallas as pl
from jax.experimental.pallas import tpu as pltpu
from jax.experimental.pallas import tpu_sc as plsc

B = 1024
D = 10
NEG = 4
HIST = 20
TL = 15
NEWS_N = NEG + 1 + HIST + D
DIM = 128
SLOT = 1 + TL

NC, NS = 2, 16
NW = NC * NS

W_CH = 224
N_CH = 56


def _make_sc_gather(nb):
    n_word = nb * NEWS_N * SLOT
    n_news = nb * NEWS_N
    n_user = nb * (1 + D)
    w_per = n_word // NW
    n_per = n_news // NW
    u_per = n_user // NW

    def body(widx, nidx, uidx, wtab, ntab, utab, wout, nout, uout,
             widx_v, wbuf, nidx_v, nbuf, uidx_v, ubuf, sem):
        wid = lax.axis_index("s") * NC + lax.axis_index("c")

        wbase = wid * w_per

        def wstep(i, carry):
            base = wbase + i * W_CH
            pltpu.sync_copy(widx.at[pl.ds(base, W_CH)], widx_v)
            pltpu.async_copy(wtab.at[widx_v], wbuf, sem).wait()
            pltpu.sync_copy(wbuf, wout.at[pl.ds(base, W_CH)])
            return carry

        lax.fori_loop(0, w_per // W_CH, wstep, 0)

        nbase = wid * n_per

        def nstep(i, carry):
            base = nbase + i * N_CH
            pltpu.sync_copy(nidx.at[pl.ds(base, N_CH)], nidx_v)
            pltpu.async_copy(ntab.at[nidx_v], nbuf, sem).wait()
            pltpu.sync_copy(nbuf, nout.at[pl.ds(base, N_CH)])
            return carry

        lax.fori_loop(0, n_per // N_CH, nstep, 0)

        ubase = wid * u_per
        pltpu.sync_copy(uidx.at[pl.ds(ubase, u_per)], uidx_v)
        pltpu.async_copy(utab.at[uidx_v], ubuf, sem).wait()
        pltpu.sync_copy(ubuf, uout.at[pl.ds(ubase, u_per)])

    return functools.partial(
        pl.kernel,
        out_type=[
            jax.ShapeDtypeStruct((n_word, DIM), jnp.float32),
            jax.ShapeDtypeStruct((n_news, DIM), jnp.float32),
            jax.ShapeDtypeStruct((n_user, DIM), jnp.float32),
        ],
        mesh=plsc.VectorSubcoreMesh(
            core_axis_name="c", subcore_axis_name="s",
            num_cores=NC, num_subcores=NS),
        scratch_types=[
            pltpu.VMEM((W_CH,), jnp.int32),
            pltpu.VMEM((W_CH, DIM), jnp.float32),
            pltpu.VMEM((N_CH,), jnp.int32),
            pltpu.VMEM((N_CH, DIM), jnp.float32),
            pltpu.VMEM((u_per,), jnp.int32),
            pltpu.VMEM((u_per, DIM), jnp.float32),
            pltpu.SemaphoreType.DMA,
        ],
    )(body)


BB = 16
IB = BB * NEWS_N
TR = IB * SLOT
NG = IB // 8
UB = BB * (1 + D)

_INV_SQRT_D = 1.0 / math.sqrt(DIM)


def _tc_body(w_ref, n_ref, u_ref, wq_ref, wk_ref, wv_ref, qp_ref, bias_ref,
             seg_ref, out_ref, q_s, k_s, v_s, p_s, s_s, info_s):
    w = w_ref[...].astype(jnp.bfloat16)
    wq = (wq_ref[...] * _INV_SQRT_D).astype(jnp.bfloat16)
    wk = wk_ref[...].astype(jnp.bfloat16)
    wv = wv_ref[...].astype(jnp.bfloat16)
    q_s[...] = jnp.dot(w, wq,
                       preferred_element_type=jnp.float32).astype(jnp.bfloat16)
    k_s[...] = jnp.dot(w, wk,
                       preferred_element_type=jnp.float32).astype(jnp.bfloat16)
    v_s[...] = jnp.dot(w, wv,
                       preferred_element_type=jnp.float32).astype(jnp.bfloat16)
    bias = bias_ref[...]
    qp = qp_ref[...]

    def smm(g, carry):
        qg = q_s[pl.ds(g * 128, 128), :]
        kg = k_s[pl.ds(g * 128, 128), :]
        s_s[pl.ds(g * 128, 128), :] = lax.dot_general(
            qg, kg, (((1,), (1,)), ((), ())),
            preferred_element_type=jnp.float32)
        return carry

    lax.fori_loop(0, NG, smm, 0, unroll=5)

    pe = jnp.exp(s_s[...].reshape(NG, 128, 128) + bias[None, :, :])
    rec = 1.0 / jnp.sum(pe, axis=2, keepdims=True)
    p_s[...] = (pe * rec).astype(jnp.bfloat16).reshape(TR, DIM)

    p3 = p_s[...].reshape(NG, 128, 128)
    v4 = v_s[...].reshape(NG, 128, DIM)
    qp_b = jnp.broadcast_to(qp.astype(jnp.bfloat16)[None], (NG, 1, DIM))
    u_b = lax.dot_general(qp_b, v4, (((2,), (2,)), ((0,), (0,))),
                          preferred_element_type=jnp.float32)
    ps = lax.dot_general(u_b.astype(jnp.bfloat16), p3,
                         (((2,), (2,)), ((0,), (0,))),
                         preferred_element_type=jnp.float32
                         ).reshape(NG, 128)
    lane = lax.broadcasted_iota(jnp.int32, (NG, 128), 1)
    ae = jnp.exp(jnp.where(lane % SLOT != 0, ps, -1e30))
    den = jnp.dot(ae.astype(jnp.bfloat16), seg_ref[...],
                  preferred_element_type=jnp.float32)
    alpha = (ae / den).astype(jnp.bfloat16)
    blk = lax.broadcasted_iota(jnp.int32, (8, 128), 0)
    lane8 = lax.broadcasted_iota(jnp.int32, (8, 128), 1)
    bmask = (lane8 // SLOT == blk).astype(jnp.bfloat16)
    w_alpha = alpha[:, None, :] * bmask[None]
    c_b = lax.dot_general(w_alpha, p3, (((2,), (1,)), ((0,), (0,))),
                          preferred_element_type=jnp.float32)
    info_s[...] = lax.dot_general(
        c_b.astype(jnp.bfloat16), v4, (((2,), (1,)), ((0,), (0,))),
        preferred_element_type=jnp.float32).reshape(IB, DIM)

    x = info_s[...] + n_ref[...]

    r2 = lax.broadcasted_iota(jnp.int32, (BB, IB), 0)
    c2 = lax.broadcasted_iota(jnp.int32, (BB, IB), 1)
    j = c2 - r2 * NEWS_N
    wnews = jnp.where((j >= NEG + 1) & (j < NEG + 1 + HIST), 1.0 / HIST,
                      jnp.where((j >= NEG + 1 + HIST) & (j < NEWS_N),
                                1.0 / D, 0.0))
    user_vec = jnp.dot(wnews, x, preferred_element_type=jnp.float32)

    r3 = lax.broadcasted_iota(jnp.int32, (BB, UB), 0)
    c3 = lax.broadcasted_iota(jnp.int32, (BB, UB), 1)
    ju = c3 - r3 * (1 + D)
    wuser = jnp.where(ju == 0, 1.0,
                      jnp.where((ju >= 1) & (ju < 1 + D), 1.0 / D, 0.0))
    user_vec = user_vec + jnp.dot(wuser, u_ref[...],
                                  preferred_element_type=jnp.float32)

    cand = x.reshape(BB, NEWS_N, DIM)[:, :NEG + 1, :]
    logits = jnp.sum(user_vec[:, None, :] * cand, axis=2)
    out_ref[...] = logits


def _attn_bias():
    r = jnp.arange(128)[:, None]
    c = jnp.arange(128)[None, :]
    valid = ((r // SLOT) == (c // SLOT)) & ((c % SLOT) != 0)
    return jnp.where(valid, 0.0, -1e30).astype(jnp.float32)


def _seg_mat():
    r = jnp.arange(128)[:, None]
    c = jnp.arange(128)[None, :]
    return ((r // SLOT) == (c // SLOT)).astype(jnp.bfloat16)


def _tc_forward(wrows, nrows, urows, Wq, Wk, Wv, q_pool):
    nb = urows.shape[0] // (1 + D)
    grid = (nb // BB,)
    return pl.pallas_call(
        _tc_body,
        grid=grid,
        in_specs=[
            pl.BlockSpec((TR, DIM), lambda i: (i, 0)),
            pl.BlockSpec((IB, DIM), lambda i: (i, 0)),
            pl.BlockSpec((UB, DIM), lambda i: (i, 0)),
            pl.BlockSpec((DIM, DIM), lambda i: (0, 0)),
            pl.BlockSpec((DIM, DIM), lambda i: (0, 0)),
            pl.BlockSpec((DIM, DIM), lambda i: (0, 0)),
            pl.BlockSpec((1, DIM), lambda i: (0, 0)),
            pl.BlockSpec((128, 128), lambda i: (0, 0)),
            pl.BlockSpec((128, 128), lambda i: (0, 0)),
        ],
        out_specs=pl.BlockSpec((BB, NEG + 1), lambda i: (i, 0)),
        out_shape=jax.ShapeDtypeStruct((nb, NEG + 1), jnp.float32),
        scratch_shapes=[
            pltpu.VMEM((TR, DIM), jnp.bfloat16),
            pltpu.VMEM((TR, DIM), jnp.bfloat16),
            pltpu.VMEM((TR, DIM), jnp.bfloat16),
            pltpu.VMEM((TR, DIM), jnp.bfloat16),
            pltpu.VMEM((TR, DIM), jnp.float32),
            pltpu.VMEM((IB, DIM), jnp.float32),
        ],
    )(wrows, nrows, urows, Wq, Wk, Wv, q_pool.reshape(1, DIM), _attn_bias(),
      _seg_mat())


def kernel(data, user_emb, news_emb, word_emb, Wq, Wk, Wv, q_pool):
    nb = B // 4
    sc = _make_sc_gather(nb)
    halves = []
    for h in range(4):
        d = data[h * nb:(h + 1) * nb]
        uidx = d[:, : 1 + D].reshape(-1)
        nidx = d[:, 1 + D: 1 + D + NEWS_N].reshape(-1)
        widx = d[:, 1 + D + NEWS_N:].reshape(-1)
        halves.append(sc(widx, nidx, uidx, word_emb, news_emb, user_emb))
    outs = [_tc_forward(wr, nr, ur, Wq, Wk, Wv, q_pool)
            for wr, nr, ur in halves]
    return jnp.concatenate(outs, axis=0)


# speedup vs baseline: 1.5149x; 1.0884x over previous
import functools
import math

import jax
import jax.numpy as jnp
from jax import lax
from jax.experimental import pallas as pl
from jax.experimental.pallas import tpu as pltpu
from jax.experimental.pallas import tpu_sc as plsc

B = 1024
D = 10
NEG = 4
HIST = 20
TL = 15
NEWS_N = NEG + 1 + HIST + D
DIM = 128
SLOT = 1 + TL

NC, NS = 2, 16
NW = NC * NS

W_CH = 224
N_CH = 56


def _make_sc_gather(nb):
    n_word = nb * NEWS_N * SLOT
    n_news = nb * NEWS_N
    n_user = nb * (1 + D)
    w_per = n_word // NW
    n_per = n_news // NW
    u_per = n_user // NW

    def body(widx, nidx, uidx, wtab, ntab, utab, wout, nout, uout,
             widx_v, wbuf, nidx_v, nbuf, uidx_v, ubuf, sem):
        wid = lax.axis_index("s") * NC + lax.axis_index("c")

        wbase = wid * w_per

        def wstep(i, carry):
            base = wbase + i * W_CH
            pltpu.sync_copy(widx.at[pl.ds(base, W_CH)], widx_v)
            pltpu.async_copy(wtab.at[widx_v], wbuf, sem).wait()
            pltpu.sync_copy(wbuf, wout.at[pl.ds(base, W_CH)])
            return carry

        lax.fori_loop(0, w_per // W_CH, wstep, 0)

        nbase = wid * n_per

        def nstep(i, carry):
            base = nbase + i * N_CH
            pltpu.sync_copy(nidx.at[pl.ds(base, N_CH)], nidx_v)
            pltpu.async_copy(ntab.at[nidx_v], nbuf, sem).wait()
            pltpu.sync_copy(nbuf, nout.at[pl.ds(base, N_CH)])
            return carry

        lax.fori_loop(0, n_per // N_CH, nstep, 0)

        ubase = wid * u_per
        pltpu.sync_copy(uidx.at[pl.ds(ubase, u_per)], uidx_v)
        pltpu.async_copy(utab.at[uidx_v], ubuf, sem).wait()
        pltpu.sync_copy(ubuf, uout.at[pl.ds(ubase, u_per)])

    return functools.partial(
        pl.kernel,
        out_type=[
            jax.ShapeDtypeStruct((n_word, DIM), jnp.float32),
            jax.ShapeDtypeStruct((n_news, DIM), jnp.float32),
            jax.ShapeDtypeStruct((n_user, DIM), jnp.float32),
        ],
        mesh=plsc.VectorSubcoreMesh(
            core_axis_name="c", subcore_axis_name="s",
            num_cores=NC, num_subcores=NS),
        scratch_types=[
            pltpu.VMEM((W_CH,), jnp.int32),
            pltpu.VMEM((W_CH, DIM), jnp.float32),
            pltpu.VMEM((N_CH,), jnp.int32),
            pltpu.VMEM((N_CH, DIM), jnp.float32),
            pltpu.VMEM((u_per,), jnp.int32),
            pltpu.VMEM((u_per, DIM), jnp.float32),
            pltpu.SemaphoreType.DMA,
        ],
    )(body)


BB = 16
IB = BB * NEWS_N
TR = IB * SLOT
NG = IB // 8
UB = BB * (1 + D)

_INV_SQRT_D = 1.0 / math.sqrt(DIM)


def _tc_body(w_ref, n_ref, u_ref, wq_ref, wk_ref, wv_ref, qp_ref, bias_ref,
             seg_ref, out_ref, q_s, k_s, v_s, p_s, s_s, info_s):
    w = w_ref[...].astype(jnp.bfloat16)
    wq = (wq_ref[...] * _INV_SQRT_D).astype(jnp.bfloat16)
    wk = wk_ref[...].astype(jnp.bfloat16)
    wv = wv_ref[...].astype(jnp.bfloat16)
    q_s[...] = jnp.dot(w, wq,
                       preferred_element_type=jnp.float32).astype(jnp.bfloat16)
    k_s[...] = jnp.dot(w, wk,
                       preferred_element_type=jnp.float32).astype(jnp.bfloat16)
    v_s[...] = jnp.dot(w, wv,
                       preferred_element_type=jnp.float32).astype(jnp.bfloat16)
    bias = bias_ref[...]
    qp = qp_ref[...]

    def smm(g, carry):
        qg = q_s[pl.ds(g * 128, 128), :]
        kg = k_s[pl.ds(g * 128, 128), :]
        s_s[pl.ds(g * 128, 128), :] = lax.dot_general(
            qg, kg, (((1,), (1,)), ((), ())),
            preferred_element_type=jnp.float32)
        return carry

    lax.fori_loop(0, NG, smm, 0, unroll=10)

    pe = jnp.exp(s_s[...].reshape(NG, 128, 128) + bias[None, :, :])
    rec = 1.0 / jnp.sum(pe, axis=2, keepdims=True)
    p_s[...] = (pe * rec).astype(jnp.bfloat16).reshape(TR, DIM)

    p3 = p_s[...].reshape(NG, 128, 128)
    v4 = v_s[...].reshape(NG, 128, DIM)
    qp_b = jnp.broadcast_to(qp.astype(jnp.bfloat16)[None], (NG, 1, DIM))
    u_b = lax.dot_general(qp_b, v4, (((2,), (2,)), ((0,), (0,))),
                          preferred_element_type=jnp.float32)
    ps = lax.dot_general(u_b.astype(jnp.bfloat16), p3,
                         (((2,), (2,)), ((0,), (0,))),
                         preferred_element_type=jnp.float32
                         ).reshape(NG, 128)
    lane = lax.broadcasted_iota(jnp.int32, (NG, 128), 1)
    ae = jnp.exp(jnp.where(lane % SLOT != 0, ps, -1e30))
    den = jnp.dot(ae.astype(jnp.bfloat16), seg_ref[...],
                  preferred_element_type=jnp.float32)
    alpha = (ae / den).astype(jnp.bfloat16)
    blk = lax.broadcasted_iota(jnp.int32, (8, 128), 0)
    lane8 = lax.broadcasted_iota(jnp.int32, (8, 128), 1)
    bmask = (lane8 // SLOT == blk).astype(jnp.bfloat16)
    w_alpha = alpha[:, None, :] * bmask[None]
    c_b = lax.dot_general(w_alpha, p3, (((2,), (1,)), ((0,), (0,))),
                          preferred_element_type=jnp.float32)
    info_s[...] = lax.dot_general(
        c_b.astype(jnp.bfloat16), v4, (((2,), (1,)), ((0,), (0,))),
        preferred_element_type=jnp.float32).reshape(IB, DIM)

    x = info_s[...] + n_ref[...]

    r2 = lax.broadcasted_iota(jnp.int32, (BB, IB), 0)
    c2 = lax.broadcasted_iota(jnp.int32, (BB, IB), 1)
    j = c2 - r2 * NEWS_N
    wnews = jnp.where((j >= NEG + 1) & (j < NEG + 1 + HIST), 1.0 / HIST,
                      jnp.where((j >= NEG + 1 + HIST) & (j < NEWS_N),
                                1.0 / D, 0.0))
    user_vec = jnp.dot(wnews, x, preferred_element_type=jnp.float32)

    r3 = lax.broadcasted_iota(jnp.int32, (BB, UB), 0)
    c3 = lax.broadcasted_iota(jnp.int32, (BB, UB), 1)
    ju = c3 - r3 * (1 + D)
    wuser = jnp.where(ju == 0, 1.0,
                      jnp.where((ju >= 1) & (ju < 1 + D), 1.0 / D, 0.0))
    user_vec = user_vec + jnp.dot(wuser, u_ref[...],
                                  preferred_element_type=jnp.float32)

    cand = x.reshape(BB, NEWS_N, DIM)[:, :NEG + 1, :]
    logits = jnp.sum(user_vec[:, None, :] * cand, axis=2)
    out_ref[...] = logits


def _attn_bias():
    r = jnp.arange(128)[:, None]
    c = jnp.arange(128)[None, :]
    valid = ((r // SLOT) == (c // SLOT)) & ((c % SLOT) != 0)
    return jnp.where(valid, 0.0, -1e30).astype(jnp.float32)


def _seg_mat():
    r = jnp.arange(128)[:, None]
    c = jnp.arange(128)[None, :]
    return ((r // SLOT) == (c // SLOT)).astype(jnp.bfloat16)


def _tc_forward(wrows, nrows, urows, Wq, Wk, Wv, q_pool):
    nb = urows.shape[0] // (1 + D)
    grid = (nb // BB,)
    return pl.pallas_call(
        _tc_body,
        grid=grid,
        in_specs=[
            pl.BlockSpec((TR, DIM), lambda i: (i, 0)),
            pl.BlockSpec((IB, DIM), lambda i: (i, 0)),
            pl.BlockSpec((UB, DIM), lambda i: (i, 0)),
            pl.BlockSpec((DIM, DIM), lambda i: (0, 0)),
            pl.BlockSpec((DIM, DIM), lambda i: (0, 0)),
            pl.BlockSpec((DIM, DIM), lambda i: (0, 0)),
            pl.BlockSpec((1, DIM), lambda i: (0, 0)),
            pl.BlockSpec((128, 128), lambda i: (0, 0)),
            pl.BlockSpec((128, 128), lambda i: (0, 0)),
        ],
        out_specs=pl.BlockSpec((BB, NEG + 1), lambda i: (i, 0)),
        out_shape=jax.ShapeDtypeStruct((nb, NEG + 1), jnp.float32),
        scratch_shapes=[
            pltpu.VMEM((TR, DIM), jnp.bfloat16),
            pltpu.VMEM((TR, DIM), jnp.bfloat16),
            pltpu.VMEM((TR, DIM), jnp.bfloat16),
            pltpu.VMEM((TR, DIM), jnp.bfloat16),
            pltpu.VMEM((TR, DIM), jnp.float32),
            pltpu.VMEM((IB, DIM), jnp.float32),
        ],
    )(wrows, nrows, urows, Wq, Wk, Wv, q_pool.reshape(1, DIM), _attn_bias(),
      _seg_mat())


def kernel(data, user_emb, news_emb, word_emb, Wq, Wk, Wv, q_pool):
    nb = B // 4
    sc = _make_sc_gather(nb)
    halves = []
    for h in range(4):
        d = data[h * nb:(h + 1) * nb]
        uidx = d[:, : 1 + D].reshape(-1)
        nidx = d[:, 1 + D: 1 + D + NEWS_N].reshape(-1)
        widx = d[:, 1 + D + NEWS_N:].reshape(-1)
        halves.append(sc(widx, nidx, uidx, word_emb, news_emb, user_emb))
    outs = [_tc_forward(wr, nr, ur, Wq, Wk, Wv, q_pool)
            for wr, nr, ur in halves]
    return jnp.concatenate(outs, axis=0)


# speedup vs baseline: 1.5501x; 1.0232x over previous
import functools
import math

import jax
import jax.numpy as jnp
from jax import lax
from jax.experimental import pallas as pl
from jax.experimental.pallas import tpu as pltpu
from jax.experimental.pallas import tpu_sc as plsc

B = 1024
D = 10
NEG = 4
HIST = 20
TL = 15
NEWS_N = NEG + 1 + HIST + D
DIM = 128
SLOT = 1 + TL

NC, NS = 2, 16
NW = NC * NS

W_CH = 224
N_CH = 56


def _make_sc_gather(nb):
    n_word = nb * NEWS_N * SLOT
    n_news = nb * NEWS_N
    n_user = nb * (1 + D)
    w_per = n_word // NW
    n_per = n_news // NW
    u_per = n_user // NW

    def body(widx, nidx, uidx, wtab, ntab, utab, wout, nout, uout,
             widx_v, wbuf, nidx_v, nbuf, uidx_v, ubuf, sem):
        wid = lax.axis_index("s") * NC + lax.axis_index("c")

        wbase = wid * w_per

        def wstep(i, carry):
            base = wbase + i * W_CH
            pltpu.sync_copy(widx.at[pl.ds(base, W_CH)], widx_v)
            pltpu.async_copy(wtab.at[widx_v], wbuf, sem).wait()
            pltpu.sync_copy(wbuf, wout.at[pl.ds(base, W_CH)])
            return carry

        lax.fori_loop(0, w_per // W_CH, wstep, 0)

        nbase = wid * n_per

        def nstep(i, carry):
            base = nbase + i * N_CH
            pltpu.sync_copy(nidx.at[pl.ds(base, N_CH)], nidx_v)
            pltpu.async_copy(ntab.at[nidx_v], nbuf, sem).wait()
            pltpu.sync_copy(nbuf, nout.at[pl.ds(base, N_CH)])
            return carry

        lax.fori_loop(0, n_per // N_CH, nstep, 0)

        ubase = wid * u_per
        pltpu.sync_copy(uidx.at[pl.ds(ubase, u_per)], uidx_v)
        pltpu.async_copy(utab.at[uidx_v], ubuf, sem).wait()
        pltpu.sync_copy(ubuf, uout.at[pl.ds(ubase, u_per)])

    return functools.partial(
        pl.kernel,
        out_type=[
            jax.ShapeDtypeStruct((n_word, DIM), jnp.float32),
            jax.ShapeDtypeStruct((n_news, DIM), jnp.float32),
            jax.ShapeDtypeStruct((n_user, DIM), jnp.float32),
        ],
        mesh=plsc.VectorSubcoreMesh(
            core_axis_name="c", subcore_axis_name="s",
            num_cores=NC, num_subcores=NS),
        scratch_types=[
            pltpu.VMEM((W_CH,), jnp.int32),
            pltpu.VMEM((W_CH, DIM), jnp.float32),
            pltpu.VMEM((N_CH,), jnp.int32),
            pltpu.VMEM((N_CH, DIM), jnp.float32),
            pltpu.VMEM((u_per,), jnp.int32),
            pltpu.VMEM((u_per, DIM), jnp.float32),
            pltpu.SemaphoreType.DMA,
        ],
    )(body)


BB = 16
IB = BB * NEWS_N
TR = IB * SLOT
NG = IB // 8
UB = BB * (1 + D)

_INV_SQRT_D = 1.0 / math.sqrt(DIM)


def _tc_body(w_ref, n_ref, u_ref, wq_ref, wk_ref, wv_ref, qp_ref, bias_ref,
             seg_ref, out_ref, q_s, k_s, v_s, p_s, s_s, info_s):
    w = w_ref[...].astype(jnp.bfloat16)
    wq = (wq_ref[...] * _INV_SQRT_D).astype(jnp.bfloat16)
    wk = wk_ref[...].astype(jnp.bfloat16)
    wv = wv_ref[...].astype(jnp.bfloat16)
    q_s[...] = jnp.dot(w, wq,
                       preferred_element_type=jnp.float32).astype(jnp.bfloat16)
    k_s[...] = jnp.dot(w, wk,
                       preferred_element_type=jnp.float32).astype(jnp.bfloat16)
    v_s[...] = jnp.dot(w, wv,
                       preferred_element_type=jnp.float32).astype(jnp.bfloat16)
    bias = bias_ref[...]
    qp = qp_ref[...]

    def smm(g, carry):
        qg = q_s[pl.ds(g * 128, 128), :]
        kg = k_s[pl.ds(g * 128, 128), :]
        s_s[pl.ds(g * 128, 128), :] = lax.dot_general(
            qg, kg, (((1,), (1,)), ((), ())),
            preferred_element_type=jnp.float32)
        return carry

    lax.fori_loop(0, NG, smm, 0, unroll=14)

    pe = jnp.exp(s_s[...].reshape(NG, 128, 128) + bias[None, :, :])
    rec = 1.0 / jnp.sum(pe, axis=2, keepdims=True)
    p_s[...] = (pe * rec).astype(jnp.bfloat16).reshape(TR, DIM)

    p3 = p_s[...].reshape(NG, 128, 128)
    v4 = v_s[...].reshape(NG, 128, DIM)
    qp_b = jnp.broadcast_to(qp.astype(jnp.bfloat16)[None], (NG, 1, DIM))
    u_b = lax.dot_general(qp_b, v4, (((2,), (2,)), ((0,), (0,))),
                          preferred_element_type=jnp.float32)
    ps = lax.dot_general(u_b.astype(jnp.bfloat16), p3,
                         (((2,), (2,)), ((0,), (0,))),
                         preferred_element_type=jnp.float32
                         ).reshape(NG, 128)
    lane = lax.broadcasted_iota(jnp.int32, (NG, 128), 1)
    ae = jnp.exp(jnp.where(lane % SLOT != 0, ps, -1e30))
    den = jnp.dot(ae.astype(jnp.bfloat16), seg_ref[...],
                  preferred_element_type=jnp.float32)
    alpha = (ae / den).astype(jnp.bfloat16)
    blk = lax.broadcasted_iota(jnp.int32, (8, 128), 0)
    lane8 = lax.broadcasted_iota(jnp.int32, (8, 128), 1)
    bmask = (lane8 // SLOT == blk).astype(jnp.bfloat16)
    w_alpha = alpha[:, None, :] * bmask[None]
    c_b = lax.dot_general(w_alpha, p3, (((2,), (1,)), ((0,), (0,))),
                          preferred_element_type=jnp.float32)
    info_s[...] = lax.dot_general(
        c_b.astype(jnp.bfloat16), v4, (((2,), (1,)), ((0,), (0,))),
        preferred_element_type=jnp.float32).reshape(IB, DIM)

    x = info_s[...] + n_ref[...]

    r2 = lax.broadcasted_iota(jnp.int32, (BB, IB), 0)
    c2 = lax.broadcasted_iota(jnp.int32, (BB, IB), 1)
    j = c2 - r2 * NEWS_N
    wnews = jnp.where((j >= NEG + 1) & (j < NEG + 1 + HIST), 1.0 / HIST,
                      jnp.where((j >= NEG + 1 + HIST) & (j < NEWS_N),
                                1.0 / D, 0.0))
    user_vec = jnp.dot(wnews, x, preferred_element_type=jnp.float32)

    r3 = lax.broadcasted_iota(jnp.int32, (BB, UB), 0)
    c3 = lax.broadcasted_iota(jnp.int32, (BB, UB), 1)
    ju = c3 - r3 * (1 + D)
    wuser = jnp.where(ju == 0, 1.0,
                      jnp.where((ju >= 1) & (ju < 1 + D), 1.0 / D, 0.0))
    user_vec = user_vec + jnp.dot(wuser, u_ref[...],
                                  preferred_element_type=jnp.float32)

    cand = x.reshape(BB, NEWS_N, DIM)[:, :NEG + 1, :]
    logits = jnp.sum(user_vec[:, None, :] * cand, axis=2)
    out_ref[...] = logits


def _attn_bias():
    r = jnp.arange(128)[:, None]
    c = jnp.arange(128)[None, :]
    valid = ((r // SLOT) == (c // SLOT)) & ((c % SLOT) != 0)
    return jnp.where(valid, 0.0, -1e30).astype(jnp.float32)


def _seg_mat():
    r = jnp.arange(128)[:, None]
    c = jnp.arange(128)[None, :]
    return ((r // SLOT) == (c // SLOT)).astype(jnp.bfloat16)


def _tc_forward(wrows, nrows, urows, Wq, Wk, Wv, q_pool):
    nb = urows.shape[0] // (1 + D)
    grid = (nb // BB,)
    return pl.pallas_call(
        _tc_body,
        grid=grid,
        in_specs=[
            pl.BlockSpec((TR, DIM), lambda i: (i, 0)),
            pl.BlockSpec((IB, DIM), lambda i: (i, 0)),
            pl.BlockSpec((UB, DIM), lambda i: (i, 0)),
            pl.BlockSpec((DIM, DIM), lambda i: (0, 0)),
            pl.BlockSpec((DIM, DIM), lambda i: (0, 0)),
            pl.BlockSpec((DIM, DIM), lambda i: (0, 0)),
            pl.BlockSpec((1, DIM), lambda i: (0, 0)),
            pl.BlockSpec((128, 128), lambda i: (0, 0)),
            pl.BlockSpec((128, 128), lambda i: (0, 0)),
        ],
        out_specs=pl.BlockSpec((BB, NEG + 1), lambda i: (i, 0)),
        out_shape=jax.ShapeDtypeStruct((nb, NEG + 1), jnp.float32),
        scratch_shapes=[
            pltpu.VMEM((TR, DIM), jnp.bfloat16),
            pltpu.VMEM((TR, DIM), jnp.bfloat16),
            pltpu.VMEM((TR, DIM), jnp.bfloat16),
            pltpu.VMEM((TR, DIM), jnp.bfloat16),
            pltpu.VMEM((TR, DIM), jnp.float32),
            pltpu.VMEM((IB, DIM), jnp.float32),
        ],
    )(wrows, nrows, urows, Wq, Wk, Wv, q_pool.reshape(1, DIM), _attn_bias(),
      _seg_mat())


def kernel(data, user_emb, news_emb, word_emb, Wq, Wk, Wv, q_pool):
    nb = B // 4
    sc = _make_sc_gather(nb)
    halves = []
    for h in range(4):
        d = data[h * nb:(h + 1) * nb]
        uidx = d[:, : 1 + D].reshape(-1)
        nidx = d[:, 1 + D: 1 + D + NEWS_N].reshape(-1)
        widx = d[:, 1 + D + NEWS_N:].reshape(-1)
        halves.append(sc(widx, nidx, uidx, word_emb, news_emb, user_emb))
    outs = [_tc_forward(wr, nr, ur, Wq, Wk, Wv, q_pool)
            for wr, nr, ur in halves]
    return jnp.concatenate(outs, axis=0)


# speedup vs baseline: 1.6173x; 1.0434x over previous
import functools
import math

import jax
import jax.numpy as jnp
from jax import lax
from jax.experimental import pallas as pl
from jax.experimental.pallas import tpu as pltpu
from jax.experimental.pallas import tpu_sc as plsc

B = 1024
D = 10
NEG = 4
HIST = 20
TL = 15
NEWS_N = NEG + 1 + HIST + D
DIM = 128
SLOT = 1 + TL

NC, NS = 2, 16
NW = NC * NS

W_CH = 224
N_CH = 56


def _make_sc_gather(nb):
    n_word = nb * NEWS_N * SLOT
    n_news = nb * NEWS_N
    n_user = nb * (1 + D)
    w_per = n_word // NW
    n_per = n_news // NW
    u_per = n_user // NW

    def body(widx, nidx, uidx, wtab, ntab, utab, wout, nout, uout,
             widx_v, wbuf, nidx_v, nbuf, uidx_v, ubuf, sem):
        wid = lax.axis_index("s") * NC + lax.axis_index("c")

        wbase = wid * w_per

        def wstep(i, carry):
            base = wbase + i * W_CH
            pltpu.sync_copy(widx.at[pl.ds(base, W_CH)], widx_v)
            pltpu.async_copy(wtab.at[widx_v], wbuf, sem).wait()
            pltpu.sync_copy(wbuf, wout.at[pl.ds(base, W_CH)])
            return carry

        lax.fori_loop(0, w_per // W_CH, wstep, 0)

        nbase = wid * n_per

        def nstep(i, carry):
            base = nbase + i * N_CH
            pltpu.sync_copy(nidx.at[pl.ds(base, N_CH)], nidx_v)
            pltpu.async_copy(ntab.at[nidx_v], nbuf, sem).wait()
            pltpu.sync_copy(nbuf, nout.at[pl.ds(base, N_CH)])
            return carry

        lax.fori_loop(0, n_per // N_CH, nstep, 0)

        ubase = wid * u_per
        pltpu.sync_copy(uidx.at[pl.ds(ubase, u_per)], uidx_v)
        pltpu.async_copy(utab.at[uidx_v], ubuf, sem).wait()
        pltpu.sync_copy(ubuf, uout.at[pl.ds(ubase, u_per)])

    return functools.partial(
        pl.kernel,
        out_type=[
            jax.ShapeDtypeStruct((n_word, DIM), jnp.float32),
            jax.ShapeDtypeStruct((n_news, DIM), jnp.float32),
            jax.ShapeDtypeStruct((n_user, DIM), jnp.float32),
        ],
        mesh=plsc.VectorSubcoreMesh(
            core_axis_name="c", subcore_axis_name="s",
            num_cores=NC, num_subcores=NS),
        scratch_types=[
            pltpu.VMEM((W_CH,), jnp.int32),
            pltpu.VMEM((W_CH, DIM), jnp.float32),
            pltpu.VMEM((N_CH,), jnp.int32),
            pltpu.VMEM((N_CH, DIM), jnp.float32),
            pltpu.VMEM((u_per,), jnp.int32),
            pltpu.VMEM((u_per, DIM), jnp.float32),
            pltpu.SemaphoreType.DMA,
        ],
    )(body)


BB = 16
IB = BB * NEWS_N
TR = IB * SLOT
NG = IB // 8
UB = BB * (1 + D)

_INV_SQRT_D = 1.0 / math.sqrt(DIM)


def _tc_body(w_ref, n_ref, u_ref, wq_ref, wk_ref, wv_ref, qp_ref, bias_ref,
             seg_ref, out_ref, q_s, k_s, v_s, p_s, s_s, info_s):
    w = w_ref[...].astype(jnp.bfloat16)
    wq = (wq_ref[...] * _INV_SQRT_D).astype(jnp.bfloat16)
    wk = wk_ref[...].astype(jnp.bfloat16)
    wv = wv_ref[...].astype(jnp.bfloat16)
    q_s[...] = jnp.dot(w, wq,
                       preferred_element_type=jnp.float32).astype(jnp.bfloat16)
    k_s[...] = jnp.dot(w, wk,
                       preferred_element_type=jnp.float32).astype(jnp.bfloat16)
    v_s[...] = jnp.dot(w, wv,
                       preferred_element_type=jnp.float32).astype(jnp.bfloat16)
    bias = bias_ref[...]
    qp = qp_ref[...]

    def smm(g, carry):
        qg = q_s[pl.ds(g * 128, 128), :]
        kg = k_s[pl.ds(g * 128, 128), :]
        s_s[pl.ds(g * 128, 128), :] = lax.dot_general(
            qg, kg, (((1,), (1,)), ((), ())),
            preferred_element_type=jnp.float32)
        return carry

    lax.fori_loop(0, NG, smm, 0, unroll=35)

    pe = jnp.exp(s_s[...].reshape(NG, 128, 128) + bias[None, :, :])
    rec = 1.0 / jnp.sum(pe, axis=2, keepdims=True)
    p_s[...] = (pe * rec).astype(jnp.bfloat16).reshape(TR, DIM)

    p3 = p_s[...].reshape(NG, 128, 128)
    v4 = v_s[...].reshape(NG, 128, DIM)
    qp_b = jnp.broadcast_to(qp.astype(jnp.bfloat16)[None], (NG, 1, DIM))
    u_b = lax.dot_general(qp_b, v4, (((2,), (2,)), ((0,), (0,))),
                          preferred_element_type=jnp.float32)
    ps = lax.dot_general(u_b.astype(jnp.bfloat16), p3,
                         (((2,), (2,)), ((0,), (0,))),
                         preferred_element_type=jnp.float32
                         ).reshape(NG, 128)
    lane = lax.broadcasted_iota(jnp.int32, (NG, 128), 1)
    ae = jnp.exp(jnp.where(lane % SLOT != 0, ps, -1e30))
    den = jnp.dot(ae.astype(jnp.bfloat16), seg_ref[...],
                  preferred_element_type=jnp.float32)
    alpha = (ae / den).astype(jnp.bfloat16)
    blk = lax.broadcasted_iota(jnp.int32, (8, 128), 0)
    lane8 = lax.broadcasted_iota(jnp.int32, (8, 128), 1)
    bmask = (lane8 // SLOT == blk).astype(jnp.bfloat16)
    w_alpha = alpha[:, None, :] * bmask[None]
    c_b = lax.dot_general(w_alpha, p3, (((2,), (1,)), ((0,), (0,))),
                          preferred_element_type=jnp.float32)
    info_s[...] = lax.dot_general(
        c_b.astype(jnp.bfloat16), v4, (((2,), (1,)), ((0,), (0,))),
        preferred_element_type=jnp.float32).reshape(IB, DIM)

    x = info_s[...] + n_ref[...]

    r2 = lax.broadcasted_iota(jnp.int32, (BB, IB), 0)
    c2 = lax.broadcasted_iota(jnp.int32, (BB, IB), 1)
    j = c2 - r2 * NEWS_N
    wnews = jnp.where((j >= NEG + 1) & (j < NEG + 1 + HIST), 1.0 / HIST,
                      jnp.where((j >= NEG + 1 + HIST) & (j < NEWS_N),
                                1.0 / D, 0.0))
    user_vec = jnp.dot(wnews, x, preferred_element_type=jnp.float32)

    r3 = lax.broadcasted_iota(jnp.int32, (BB, UB), 0)
    c3 = lax.broadcasted_iota(jnp.int32, (BB, UB), 1)
    ju = c3 - r3 * (1 + D)
    wuser = jnp.where(ju == 0, 1.0,
                      jnp.where((ju >= 1) & (ju < 1 + D), 1.0 / D, 0.0))
    user_vec = user_vec + jnp.dot(wuser, u_ref[...],
                                  preferred_element_type=jnp.float32)

    cand = x.reshape(BB, NEWS_N, DIM)[:, :NEG + 1, :]
    logits = jnp.sum(user_vec[:, None, :] * cand, axis=2)
    out_ref[...] = logits


def _attn_bias():
    r = jnp.arange(128)[:, None]
    c = jnp.arange(128)[None, :]
    valid = ((r // SLOT) == (c // SLOT)) & ((c % SLOT) != 0)
    return jnp.where(valid, 0.0, -1e30).astype(jnp.float32)


def _seg_mat():
    r = jnp.arange(128)[:, None]
    c = jnp.arange(128)[None, :]
    return ((r // SLOT) == (c // SLOT)).astype(jnp.bfloat16)


def _tc_forward(wrows, nrows, urows, Wq, Wk, Wv, q_pool):
    nb = urows.shape[0] // (1 + D)
    grid = (nb // BB,)
    return pl.pallas_call(
        _tc_body,
        grid=grid,
        in_specs=[
            pl.BlockSpec((TR, DIM), lambda i: (i, 0)),
            pl.BlockSpec((IB, DIM), lambda i: (i, 0)),
            pl.BlockSpec((UB, DIM), lambda i: (i, 0)),
            pl.BlockSpec((DIM, DIM), lambda i: (0, 0)),
            pl.BlockSpec((DIM, DIM), lambda i: (0, 0)),
            pl.BlockSpec((DIM, DIM), lambda i: (0, 0)),
            pl.BlockSpec((1, DIM), lambda i: (0, 0)),
            pl.BlockSpec((128, 128), lambda i: (0, 0)),
            pl.BlockSpec((128, 128), lambda i: (0, 0)),
        ],
        out_specs=pl.BlockSpec((BB, NEG + 1), lambda i: (i, 0)),
        out_shape=jax.ShapeDtypeStruct((nb, NEG + 1), jnp.float32),
        scratch_shapes=[
            pltpu.VMEM((TR, DIM), jnp.bfloat16),
            pltpu.VMEM((TR, DIM), jnp.bfloat16),
            pltpu.VMEM((TR, DIM), jnp.bfloat16),
            pltpu.VMEM((TR, DIM), jnp.bfloat16),
            pltpu.VMEM((TR, DIM), jnp.float32),
            pltpu.VMEM((IB, DIM), jnp.float32),
        ],
    )(wrows, nrows, urows, Wq, Wk, Wv, q_pool.reshape(1, DIM), _attn_bias(),
      _seg_mat())


def kernel(data, user_emb, news_emb, word_emb, Wq, Wk, Wv, q_pool):
    nb = B // 4
    sc = _make_sc_gather(nb)
    halves = []
    for h in range(4):
        d = data[h * nb:(h + 1) * nb]
        uidx = d[:, : 1 + D].reshape(-1)
        nidx = d[:, 1 + D: 1 + D + NEWS_N].reshape(-1)
        widx = d[:, 1 + D + NEWS_N:].reshape(-1)
        halves.append(sc(widx, nidx, uidx, word_emb, news_emb, user_emb))
    outs = [_tc_forward(wr, nr, ur, Wq, Wk, Wv, q_pool)
            for wr, nr, ur in halves]
    return jnp.concatenate(outs, axis=0)


# speedup vs baseline: 1.6793x; 1.0383x over previous
import functools
import math

import jax
import jax.numpy as jnp
from jax import lax
from jax.experimental import pallas as pl
from jax.experimental.pallas import tpu as pltpu
from jax.experimental.pallas import tpu_sc as plsc

B = 1024
D = 10
NEG = 4
HIST = 20
TL = 15
NEWS_N = NEG + 1 + HIST + D
DIM = 128
SLOT = 1 + TL

NC, NS = 2, 16
NW = NC * NS

W_CH = 224
N_CH = 56


def _make_sc_gather(nb):
    n_word = nb * NEWS_N * SLOT
    n_news = nb * NEWS_N
    n_user = nb * (1 + D)
    w_per = n_word // NW
    n_per = n_news // NW
    u_per = n_user // NW

    def body(widx, nidx, uidx, wtab, ntab, utab, wout, nout, uout,
             widx_v, wbuf, nidx_v, nbuf, uidx_v, ubuf, sem):
        wid = lax.axis_index("s") * NC + lax.axis_index("c")

        wbase = wid * w_per

        def wstep(i, carry):
            base = wbase + i * W_CH
            pltpu.sync_copy(widx.at[pl.ds(base, W_CH)], widx_v)
            pltpu.async_copy(wtab.at[widx_v], wbuf, sem).wait()
            pltpu.sync_copy(wbuf, wout.at[pl.ds(base, W_CH)])
            return carry

        lax.fori_loop(0, w_per // W_CH, wstep, 0)

        nbase = wid * n_per

        def nstep(i, carry):
            base = nbase + i * N_CH
            pltpu.sync_copy(nidx.at[pl.ds(base, N_CH)], nidx_v)
            pltpu.async_copy(ntab.at[nidx_v], nbuf, sem).wait()
            pltpu.sync_copy(nbuf, nout.at[pl.ds(base, N_CH)])
            return carry

        lax.fori_loop(0, n_per // N_CH, nstep, 0)

        ubase = wid * u_per
        pltpu.sync_copy(uidx.at[pl.ds(ubase, u_per)], uidx_v)
        pltpu.async_copy(utab.at[uidx_v], ubuf, sem).wait()
        pltpu.sync_copy(ubuf, uout.at[pl.ds(ubase, u_per)])

    return functools.partial(
        pl.kernel,
        out_type=[
            jax.ShapeDtypeStruct((n_word, DIM), jnp.float32),
            jax.ShapeDtypeStruct((n_news, DIM), jnp.float32),
            jax.ShapeDtypeStruct((n_user, DIM), jnp.float32),
        ],
        mesh=plsc.VectorSubcoreMesh(
            core_axis_name="c", subcore_axis_name="s",
            num_cores=NC, num_subcores=NS),
        scratch_types=[
            pltpu.VMEM((W_CH,), jnp.int32),
            pltpu.VMEM((W_CH, DIM), jnp.float32),
            pltpu.VMEM((N_CH,), jnp.int32),
            pltpu.VMEM((N_CH, DIM), jnp.float32),
            pltpu.VMEM((u_per,), jnp.int32),
            pltpu.VMEM((u_per, DIM), jnp.float32),
            pltpu.SemaphoreType.DMA,
        ],
    )(body)


BB = 16
IB = BB * NEWS_N
TR = IB * SLOT
NG = IB // 8
UB = BB * (1 + D)

_INV_SQRT_D = 1.0 / math.sqrt(DIM)


def _tc_body(w_ref, n_ref, u_ref, wq_ref, wk_ref, wv_ref, qp_ref, bias_ref,
             seg_ref, out_ref, q_s, k_s, v_s, p_s, s_s, info_s):
    w = w_ref[...].astype(jnp.bfloat16)
    wq = (wq_ref[...] * _INV_SQRT_D).astype(jnp.bfloat16)
    wk = wk_ref[...].astype(jnp.bfloat16)
    wv = wv_ref[...].astype(jnp.bfloat16)
    q_s[...] = jnp.dot(w, wq,
                       preferred_element_type=jnp.float32).astype(jnp.bfloat16)
    k_s[...] = jnp.dot(w, wk,
                       preferred_element_type=jnp.float32).astype(jnp.bfloat16)
    v_s[...] = jnp.dot(w, wv,
                       preferred_element_type=jnp.float32).astype(jnp.bfloat16)
    bias = bias_ref[...]
    qp = qp_ref[...]

    def smm(g, carry):
        qg = q_s[pl.ds(g * 128, 128), :]
        kg = k_s[pl.ds(g * 128, 128), :]
        s_s[pl.ds(g * 128, 128), :] = lax.dot_general(
            qg, kg, (((1,), (1,)), ((), ())),
            preferred_element_type=jnp.float32)
        return carry

    lax.fori_loop(0, NG, smm, 0, unroll=NG)

    pe = jnp.exp(s_s[...].reshape(NG, 128, 128) + bias[None, :, :])
    rec = 1.0 / jnp.sum(pe, axis=2, keepdims=True)
    p_s[...] = (pe * rec).astype(jnp.bfloat16).reshape(TR, DIM)

    p3 = p_s[...].reshape(NG, 128, 128)
    v4 = v_s[...].reshape(NG, 128, DIM)
    qp_b = jnp.broadcast_to(qp.astype(jnp.bfloat16)[None], (NG, 1, DIM))
    u_b = lax.dot_general(qp_b, v4, (((2,), (2,)), ((0,), (0,))),
                          preferred_element_type=jnp.float32)
    ps = lax.dot_general(u_b.astype(jnp.bfloat16), p3,
                         (((2,), (2,)), ((0,), (0,))),
                         preferred_element_type=jnp.float32
                         ).reshape(NG, 128)
    lane = lax.broadcasted_iota(jnp.int32, (NG, 128), 1)
    ae = jnp.exp(jnp.where(lane % SLOT != 0, ps, -1e30))
    den = jnp.dot(ae.astype(jnp.bfloat16), seg_ref[...],
                  preferred_element_type=jnp.float32)
    alpha = (ae / den).astype(jnp.bfloat16)
    blk = lax.broadcasted_iota(jnp.int32, (8, 128), 0)
    lane8 = lax.broadcasted_iota(jnp.int32, (8, 128), 1)
    bmask = (lane8 // SLOT == blk).astype(jnp.bfloat16)
    w_alpha = alpha[:, None, :] * bmask[None]
    c_b = lax.dot_general(w_alpha, p3, (((2,), (1,)), ((0,), (0,))),
                          preferred_element_type=jnp.float32)
    info_s[...] = lax.dot_general(
        c_b.astype(jnp.bfloat16), v4, (((2,), (1,)), ((0,), (0,))),
        preferred_element_type=jnp.float32).reshape(IB, DIM)

    x = info_s[...] + n_ref[...]

    r2 = lax.broadcasted_iota(jnp.int32, (BB, IB), 0)
    c2 = lax.broadcasted_iota(jnp.int32, (BB, IB), 1)
    j = c2 - r2 * NEWS_N
    wnews = jnp.where((j >= NEG + 1) & (j < NEG + 1 + HIST), 1.0 / HIST,
                      jnp.where((j >= NEG + 1 + HIST) & (j < NEWS_N),
                                1.0 / D, 0.0))
    user_vec = jnp.dot(wnews, x, preferred_element_type=jnp.float32)

    r3 = lax.broadcasted_iota(jnp.int32, (BB, UB), 0)
    c3 = lax.broadcasted_iota(jnp.int32, (BB, UB), 1)
    ju = c3 - r3 * (1 + D)
    wuser = jnp.where(ju == 0, 1.0,
                      jnp.where((ju >= 1) & (ju < 1 + D), 1.0 / D, 0.0))
    user_vec = user_vec + jnp.dot(wuser, u_ref[...],
                                  preferred_element_type=jnp.float32)

    cand = x.reshape(BB, NEWS_N, DIM)[:, :NEG + 1, :]
    logits = jnp.sum(user_vec[:, None, :] * cand, axis=2)
    out_ref[...] = logits


def _attn_bias():
    r = jnp.arange(128)[:, None]
    c = jnp.arange(128)[None, :]
    valid = ((r // SLOT) == (c // SLOT)) & ((c % SLOT) != 0)
    return jnp.where(valid, 0.0, -1e30).astype(jnp.float32)


def _seg_mat():
    r = jnp.arange(128)[:, None]
    c = jnp.arange(128)[None, :]
    return ((r // SLOT) == (c // SLOT)).astype(jnp.bfloat16)


def _tc_forward(wrows, nrows, urows, Wq, Wk, Wv, q_pool):
    nb = urows.shape[0] // (1 + D)
    grid = (nb // BB,)
    return pl.pallas_call(
        _tc_body,
        grid=grid,
        in_specs=[
            pl.BlockSpec((TR, DIM), lambda i: (i, 0)),
            pl.BlockSpec((IB, DIM), lambda i: (i, 0)),
            pl.BlockSpec((UB, DIM), lambda i: (i, 0)),
            pl.BlockSpec((DIM, DIM), lambda i: (0, 0)),
            pl.BlockSpec((DIM, DIM), lambda i: (0, 0)),
            pl.BlockSpec((DIM, DIM), lambda i: (0, 0)),
            pl.BlockSpec((1, DIM), lambda i: (0, 0)),
            pl.BlockSpec((128, 128), lambda i: (0, 0)),
            pl.BlockSpec((128, 128), lambda i: (0, 0)),
        ],
        out_specs=pl.BlockSpec((BB, NEG + 1), lambda i: (i, 0)),
        out_shape=jax.ShapeDtypeStruct((nb, NEG + 1), jnp.float32),
        scratch_shapes=[
            pltpu.VMEM((TR, DIM), jnp.bfloat16),
            pltpu.VMEM((TR, DIM), jnp.bfloat16),
            pltpu.VMEM((TR, DIM), jnp.bfloat16),
            pltpu.VMEM((TR, DIM), jnp.bfloat16),
            pltpu.VMEM((TR, DIM), jnp.float32),
            pltpu.VMEM((IB, DIM), jnp.float32),
        ],
    )(wrows, nrows, urows, Wq, Wk, Wv, q_pool.reshape(1, DIM), _attn_bias(),
      _seg_mat())


def kernel(data, user_emb, news_emb, word_emb, Wq, Wk, Wv, q_pool):
    nb = B // 4
    sc = _make_sc_gather(nb)
    halves = []
    for h in range(4):
        d = data[h * nb:(h + 1) * nb]
        uidx = d[:, : 1 + D].reshape(-1)
        nidx = d[:, 1 + D: 1 + D + NEWS_N].reshape(-1)
        widx = d[:, 1 + D + NEWS_N:].reshape(-1)
        halves.append(sc(widx, nidx, uidx, word_emb, news_emb, user_emb))
    outs = [_tc_forward(wr, nr, ur, Wq, Wk, Wv, q_pool)
            for wr, nr, ur in halves]
    return jnp.concatenate(outs, axis=0)
